# baseline (speedup 1.0000x reference)
import functools

import jax
import jax.numpy as jnp
from jax import lax
from jax.experimental import pallas as pl
from jax.experimental.pallas import tpu as pltpu
from jax.experimental.pallas import tpu_sc as plsc

N = 10000
E = 320000
D = 128
ALPHA = 0.5

NC = 2
NS = 16
LANES = 16
NW = NC * NS

EDGES_PER_WORKER = E // NW
BATCH = 80
NBATCH = EDGES_PER_WORKER // BATCH
GROUPS = BATCH // LANES
VPR = D // LANES
NPAD = 10240
ROWS_PER_SUB = NPAD // NS


def _sc_aggregate_fn():
  mesh = plsc.VectorSubcoreMesh(core_axis_name="c", subcore_axis_name="s")

  @functools.partial(
      pl.kernel,
      out_type=jax.ShapeDtypeStruct((NC, NPAD, D), jnp.float32),
      mesh=mesh,
      compiler_params=pltpu.CompilerParams(needs_layout_passes=False),
      scratch_types=[
          pltpu.VMEM((BATCH, D), jnp.float32),
          pltpu.VMEM((BATCH, D), jnp.float32),
          pltpu.VMEM((3, BATCH), jnp.int32),
          pltpu.VMEM((3, BATCH), jnp.int32),
          pltpu.VMEM_SHARED((NPAD, D), jnp.float32),
          pltpu.SemaphoreType.DMA,
          pltpu.SemaphoreType.DMA,
          pltpu.SemaphoreType.DMA,
          pltpu.SemaphoreType.DMA,
      ],
  )
  def agg(x_hbm, epack_hbm, out_hbm,
          rows0, rows1, chunk0, chunk1, hsh, gsem0, gsem1, csem0, csem1):
    cid = lax.axis_index("c")
    sid = lax.axis_index("s")
    wid = cid * NS + sid
    bufs = (rows0, rows1)
    chunk = (chunk0, chunk1)
    gsems = (gsem0, gsem1)
    csems = (csem0, csem1)

    zeros = jnp.zeros((LANES,), jnp.float32)

    def zrow(r, carry):
      for j in range(VPR):
        rows0[r, pl.ds(j * LANES, LANES)] = zeros
      return carry

    lax.fori_loop(0, BATCH, zrow, 0)
    for t in range(ROWS_PER_SUB // BATCH):
      pltpu.sync_copy(rows0, hsh.at[pl.ds(sid * ROWS_PER_SUB + t * BATCH, BATCH)])
    plsc.subcore_barrier()

    def stage(i, b):
      pltpu.async_copy(epack_hbm.at[wid, i], chunk[b], csems[b])

    def cwait(b):
      pltpu.make_async_copy(epack_hbm.at[0, 0], chunk[b], csems[b]).wait()

    def gather(b):
      pltpu.async_copy(x_hbm.at[chunk[b].at[0]], bufs[b], gsems[b])

    def gwait(b):
      pltpu.make_async_copy(x_hbm.at[chunk[b].at[0]], bufs[b], gsems[b]).wait()

    two = jnp.full((LANES,), 2, dtype=jnp.int32)

    def scale(b):
      rows = bufs[b]

      def grp(g, carry):
        for rr in range(LANES):
          r = g * LANES + rr
          widx = jnp.full((LANES,), r, dtype=jnp.int32)
          wbits = plsc.load_gather(chunk[b], [two, widx])
          ws = 1.0 - plsc.bitcast(wbits, jnp.float32)
          for j in range(VPR):
            rows[r, pl.ds(j * LANES, LANES)] = (
                rows[r, pl.ds(j * LANES, LANES)] * ws)
        return carry

      lax.fori_loop(0, GROUPS, grp, 0)

    def scatter(b):
      pltpu.sync_copy(bufs[b], hsh.at[chunk[b].at[1]], add=True)

    stage(0, 0)
    stage(1, 1)
    cwait(0)
    gather(0)

    def pair(p, carry):
      i0 = 2 * p
      cwait(1)
      gather(1)
      gwait(0)
      scale(0)
      scatter(0)
      stage(i0 + 2, 0)
      gwait(1)
      scale(1)
      scatter(1)
      stage(i0 + 3, 1)
      cwait(0)
      gather(0)
      return carry

    lax.fori_loop(0, (NBATCH - 3) // 2, pair, 0)
    cwait(1)
    gather(1)
    gwait(0)
    scale(0)
    scatter(0)
    stage(NBATCH - 1, 0)
    gwait(1)
    scale(1)
    scatter(1)
    cwait(0)
    gather(0)
    gwait(0)
    scale(0)
    scatter(0)
    plsc.subcore_barrier()

    pltpu.sync_copy(
        hsh.at[pl.ds(sid * ROWS_PER_SUB, ROWS_PER_SUB)],
        out_hbm.at[cid, pl.ds(sid * ROWS_PER_SUB, ROWS_PER_SUB)])

  return agg


_sc_aggregate = _sc_aggregate_fn()

BLK = 400


def _tc_linear_body(p_ref, w_ref, b_ref, o_ref):
  h = p_ref[0] + p_ref[1]
  acc = lax.dot_general(h, w_ref[...], (((1,), (1,)), ((), ())),
                        preferred_element_type=jnp.float32)
  o_ref[...] = ALPHA * (acc + b_ref[...])


def _tc_linear(partials, W, b2d):
  return pl.pallas_call(
      _tc_linear_body,
      grid=(N // BLK,),
      in_specs=[
          pl.BlockSpec((NC, BLK, D), lambda i: (0, i, 0)),
          pl.BlockSpec((D, D), lambda i: (0, 0)),
          pl.BlockSpec((1, D), lambda i: (0, 0)),
      ],
      out_specs=pl.BlockSpec((BLK, D), lambda i: (i, 0)),
      out_shape=jax.ShapeDtypeStruct((N, D), jnp.float32),
  )(partials, W, b2d)


@jax.jit
def kernel(x, edge_index, edge_w, W, b):
  wbits = lax.bitcast_convert_type(edge_w, jnp.int32)
  epack = jnp.stack(
      [edge_index[0].reshape(NW, NBATCH, BATCH),
       edge_index[1].reshape(NW, NBATCH, BATCH),
       wbits.reshape(NW, NBATCH, BATCH)], axis=2)
  partials = _sc_aggregate(x, epack)
  return _tc_linear(partials, W, b.reshape(1, D))


# speedup vs baseline: 1.4225x; 1.4225x over previous
import functools

import jax
import jax.numpy as jnp
from jax import lax
from jax.experimental import pallas as pl
from jax.experimental.pallas import tpu as pltpu
from jax.experimental.pallas import tpu_sc as plsc

N = 10000
E = 320000
D = 128
ALPHA = 0.5

NC = 2
NS = 16
LANES = 16
NW = NC * NS

EDGES_PER_WORKER = E // NW
BATCH = 80
NBATCH = EDGES_PER_WORKER // BATCH
GROUPS = BATCH // LANES
VPR = D // LANES
NPAD = 10240
ROWS_PER_SUB = NPAD // NS
QUAD = (NBATCH - 5) // 4


def _sc_aggregate_fn():
  mesh = plsc.VectorSubcoreMesh(core_axis_name="c", subcore_axis_name="s")

  @functools.partial(
      pl.kernel,
      out_type=jax.ShapeDtypeStruct((NC, NPAD, D), jnp.float32),
      mesh=mesh,
      compiler_params=pltpu.CompilerParams(needs_layout_passes=False),
      scratch_types=[
          pltpu.VMEM((BATCH, D), jnp.float32),
          pltpu.VMEM((BATCH, D), jnp.float32),
          pltpu.VMEM((4, BATCH), jnp.int32),
          pltpu.VMEM((4, BATCH), jnp.int32),
          pltpu.VMEM((4, BATCH), jnp.float32),
          pltpu.VMEM_SHARED((NPAD, D), jnp.float32),
          pltpu.SemaphoreType.DMA,
          pltpu.SemaphoreType.DMA,
          pltpu.SemaphoreType.DMA,
          pltpu.SemaphoreType.DMA,
          pltpu.SemaphoreType.DMA,
          pltpu.SemaphoreType.DMA,
      ],
  )
  def agg(x_hbm, src_hbm, dst_hbm, w_hbm, out_hbm,
          rows0, rows1, schunk, dchunk, wchunk, hsh,
          gsem0, gsem1, csem0, csem1, csem2, csem3):
    cid = lax.axis_index("c")
    sid = lax.axis_index("s")
    wid = cid * NS + sid
    base0 = wid * EDGES_PER_WORKER
    bufs = (rows0, rows1)
    gsems = (gsem0, gsem1)
    csems = (csem0, csem1, csem2, csem3)

    zeros = jnp.zeros((LANES,), jnp.float32)

    def zrow(r, carry):
      for j in range(VPR):
        rows0[r, pl.ds(j * LANES, LANES)] = zeros
      return carry

    lax.fori_loop(0, BATCH, zrow, 0)
    for t in range(ROWS_PER_SUB // BATCH):
      pltpu.sync_copy(rows0, hsh.at[pl.ds(sid * ROWS_PER_SUB + t * BATCH, BATCH)])
    plsc.subcore_barrier()

    def stage(i, k):
      base = base0 + i * BATCH
      pltpu.async_copy(src_hbm.at[pl.ds(base, BATCH)], schunk.at[k], csems[k])
      pltpu.async_copy(dst_hbm.at[pl.ds(base, BATCH)], dchunk.at[k], csems[k])
      pltpu.async_copy(w_hbm.at[pl.ds(base, BATCH)], wchunk.at[k], csems[k])

    def cwait(k):
      pltpu.make_async_copy(src_hbm.at[pl.ds(0, BATCH)], schunk.at[k], csems[k]).wait()
      pltpu.make_async_copy(dst_hbm.at[pl.ds(0, BATCH)], dchunk.at[k], csems[k]).wait()
      pltpu.make_async_copy(w_hbm.at[pl.ds(0, BATCH)], wchunk.at[k], csems[k]).wait()

    def gather(k, r):
      pltpu.async_copy(x_hbm.at[schunk.at[k]], bufs[r], gsems[r])

    def gwait(k, r):
      pltpu.make_async_copy(x_hbm.at[schunk.at[k]], bufs[r], gsems[r]).wait()

    def scale(k, r):
      rows = bufs[r]
      wrow = wchunk.at[k]

      def grp(g, carry):
        for rr in range(LANES):
          rx = g * LANES + rr
          widx = jnp.full((LANES,), rx, dtype=jnp.int32)
          ws = 1.0 - plsc.load_gather(wrow, [widx])
          for j in range(VPR):
            rows[rx, pl.ds(j * LANES, LANES)] = (
                rows[rx, pl.ds(j * LANES, LANES)] * ws)
        return carry

      lax.fori_loop(0, GROUPS, grp, 0)

    def scatter(k, r):
      pltpu.sync_copy(bufs[r], hsh.at[dchunk.at[k]], add=True)

    for k in range(4):
      stage(k, k)
    cwait(0)
    gather(0, 0)

    def quad(q, carry):
      i = 4 * q
      gwait(0, 0)
      cwait(1)
      gather(1, 1)
      scale(0, 0)
      scatter(0, 0)
      stage(i + 4, 0)
      gwait(1, 1)
      cwait(2)
      gather(2, 0)
      scale(1, 1)
      scatter(1, 1)
      stage(i + 5, 1)
      gwait(2, 0)
      cwait(3)
      gather(3, 1)
      scale(2, 0)
      scatter(2, 0)
      stage(i + 6, 2)
      gwait(3, 1)
      cwait(0)
      gather(0, 0)
      scale(3, 1)
      scatter(3, 1)
      stage(i + 7, 3)
      return carry

    lax.fori_loop(0, QUAD, quad, 0)
    gwait(0, 0)
    cwait(1)
    gather(1, 1)
    scale(0, 0)
    scatter(0, 0)
    stage(NBATCH - 1, 0)
    gwait(1, 1)
    cwait(2)
    gather(2, 0)
    scale(1, 1)
    scatter(1, 1)
    gwait(2, 0)
    cwait(3)
    gather(3, 1)
    scale(2, 0)
    scatter(2, 0)
    gwait(3, 1)
    cwait(0)
    gather(0, 0)
    scale(3, 1)
    scatter(3, 1)
    gwait(0, 0)
    scale(0, 0)
    scatter(0, 0)
    plsc.subcore_barrier()

    pltpu.sync_copy(
        hsh.at[pl.ds(sid * ROWS_PER_SUB, ROWS_PER_SUB)],
        out_hbm.at[cid, pl.ds(sid * ROWS_PER_SUB, ROWS_PER_SUB)])

  return agg


_sc_aggregate = _sc_aggregate_fn()

BLK = 400


def _tc_linear_body(p_ref, w_ref, b_ref, o_ref):
  h = p_ref[0] + p_ref[1]
  acc = lax.dot_general(h, w_ref[...], (((1,), (1,)), ((), ())),
                        preferred_element_type=jnp.float32)
  o_ref[...] = ALPHA * (acc + b_ref[...])


def _tc_linear(partials, W, b2d):
  return pl.pallas_call(
      _tc_linear_body,
      grid=(N // BLK,),
      in_specs=[
          pl.BlockSpec((NC, BLK, D), lambda i: (0, i, 0)),
          pl.BlockSpec((D, D), lambda i: (0, 0)),
          pl.BlockSpec((1, D), lambda i: (0, 0)),
      ],
      out_specs=pl.BlockSpec((BLK, D), lambda i: (i, 0)),
      out_shape=jax.ShapeDtypeStruct((N, D), jnp.float32),
  )(partials, W, b2d)


@jax.jit
def kernel(x, edge_index, edge_w, W, b):
  src = edge_index[0]
  dst = edge_index[1]
  partials = _sc_aggregate(x, src, dst, edge_w)
  return _tc_linear(partials, W, b.reshape(1, D))


# speedup vs baseline: 1.4256x; 1.0022x over previous
import functools

import jax
import jax.numpy as jnp
from jax import lax
from jax.experimental import pallas as pl
from jax.experimental.pallas import tpu as pltpu
from jax.experimental.pallas import tpu_sc as plsc

N = 10000
E = 320000
D = 128
ALPHA = 0.5

NC = 2
NS = 16
LANES = 16
NW = NC * NS

EDGES_PER_WORKER = E // NW
BATCH = 80
NBATCH = EDGES_PER_WORKER // BATCH
GROUPS = BATCH // LANES
VPR = D // LANES
NPAD = 10240
ROWS_PER_SUB = NPAD // NS
QUAD = (NBATCH - 5) // 4


def _sc_aggregate_fn():
  mesh = plsc.VectorSubcoreMesh(core_axis_name="c", subcore_axis_name="s")

  @functools.partial(
      pl.kernel,
      out_type=jax.ShapeDtypeStruct((NC, NPAD, D), jnp.float32),
      mesh=mesh,
      compiler_params=pltpu.CompilerParams(needs_layout_passes=False),
      scratch_types=[
          pltpu.VMEM((BATCH, D), jnp.float32),
          pltpu.VMEM((BATCH, D), jnp.float32),
          pltpu.VMEM((4, BATCH), jnp.int32),
          pltpu.VMEM((4, BATCH), jnp.int32),
          pltpu.VMEM((4, BATCH), jnp.float32),
          pltpu.VMEM_SHARED((NPAD, D), jnp.float32),
          pltpu.SemaphoreType.DMA,
          pltpu.SemaphoreType.DMA,
          pltpu.SemaphoreType.DMA,
          pltpu.SemaphoreType.DMA,
          pltpu.SemaphoreType.DMA,
          pltpu.SemaphoreType.DMA,
          pltpu.SemaphoreType.DMA,
          pltpu.SemaphoreType.DMA,
      ],
  )
  def agg(x_hbm, src_hbm, dst_hbm, w_hbm, out_hbm,
          rows0, rows1, schunk, dchunk, wchunk, hsh,
          gsem0, gsem1, csem0, csem1, csem2, csem3, ssem0, ssem1):
    cid = lax.axis_index("c")
    sid = lax.axis_index("s")
    wid = cid * NS + sid
    base0 = wid * EDGES_PER_WORKER
    bufs = (rows0, rows1)
    gsems = (gsem0, gsem1)
    csems = (csem0, csem1, csem2, csem3)
    ssems = (ssem0, ssem1)

    zeros = jnp.zeros((LANES,), jnp.float32)

    def zrow(buf):
      def body(r, carry):
        for j in range(VPR):
          buf[r, pl.ds(j * LANES, LANES)] = zeros
        return carry
      lax.fori_loop(0, BATCH, body, 0)

    zrow(rows0)
    zrow(rows1)
    for t in range(ROWS_PER_SUB // BATCH):
      pltpu.sync_copy(rows0, hsh.at[pl.ds(sid * ROWS_PER_SUB + t * BATCH, BATCH)])

    def stage(i, k):
      base = base0 + i * BATCH
      pltpu.async_copy(src_hbm.at[pl.ds(base, BATCH)], schunk.at[k], csems[k])
      pltpu.async_copy(dst_hbm.at[pl.ds(base, BATCH)], dchunk.at[k], csems[k])
      pltpu.async_copy(w_hbm.at[pl.ds(base, BATCH)], wchunk.at[k], csems[k])

    def cwait(k):
      pltpu.make_async_copy(src_hbm.at[pl.ds(0, BATCH)], schunk.at[k], csems[k]).wait()
      pltpu.make_async_copy(dst_hbm.at[pl.ds(0, BATCH)], dchunk.at[k], csems[k]).wait()
      pltpu.make_async_copy(w_hbm.at[pl.ds(0, BATCH)], wchunk.at[k], csems[k]).wait()

    def gather(k, r):
      pltpu.async_copy(x_hbm.at[schunk.at[k]], bufs[r], gsems[r])

    def gwait(k, r):
      pltpu.make_async_copy(x_hbm.at[schunk.at[k]], bufs[r], gsems[r]).wait()

    def scale(k, r):
      rows = bufs[r]
      wrow = wchunk.at[k]

      def grp(g, carry):
        for rr in range(LANES):
          rx = g * LANES + rr
          widx = jnp.full((LANES,), rx, dtype=jnp.int32)
          ws = 1.0 - plsc.load_gather(wrow, [widx])
          for j in range(VPR):
            rows[rx, pl.ds(j * LANES, LANES)] = (
                rows[rx, pl.ds(j * LANES, LANES)] * ws)
        return carry

      lax.fori_loop(0, GROUPS, grp, 0)

    def scatter(k, r):
      pltpu.async_copy(bufs[r], hsh.at[dchunk.at[k]], ssems[r], add=True)

    def swait(k, r):
      pltpu.make_async_copy(bufs[r], hsh.at[dchunk.at[k]], ssems[r]).wait()

    stage(0, 0)
    stage(1, 1)
    stage(2, 2)
    pltpu.async_copy(dst_hbm.at[pl.ds(base0, BATCH)], dchunk.at[3], csems[3])
    pltpu.make_async_copy(dst_hbm.at[pl.ds(0, BATCH)], dchunk.at[3], csems[3]).wait()
    plsc.subcore_barrier()
    scatter(3, 1)
    cwait(0)
    gather(0, 0)

    def quad(q, carry):
      i = 4 * q
      gwait(0, 0)
      cwait(1)
      swait(3, 1)
      gather(1, 1)
      stage(i + 3, 3)
      scale(0, 0)
      scatter(0, 0)
      gwait(1, 1)
      cwait(2)
      swait(0, 0)
      gather(2, 0)
      stage(i + 4, 0)
      scale(1, 1)
      scatter(1, 1)
      gwait(2, 0)
      cwait(3)
      swait(1, 1)
      gather(3, 1)
      stage(i + 5, 1)
      scale(2, 0)
      scatter(2, 0)
      gwait(3, 1)
      cwait(0)
      swait(2, 0)
      gather(0, 0)
      stage(i + 6, 2)
      scale(3, 1)
      scatter(3, 1)
      return carry

    lax.fori_loop(0, QUAD, quad, 0)
    gwait(0, 0)
    cwait(1)
    swait(3, 1)
    gather(1, 1)
    stage(123, 3)
    scale(0, 0)
    scatter(0, 0)
    gwait(1, 1)
    cwait(2)
    swait(0, 0)
    gather(2, 0)
    stage(124, 0)
    scale(1, 1)
    scatter(1, 1)
    gwait(2, 0)
    cwait(3)
    swait(1, 1)
    gather(3, 1)
    scale(2, 0)
    scatter(2, 0)
    gwait(3, 1)
    cwait(0)
    swait(2, 0)
    gather(0, 0)
    scale(3, 1)
    scatter(3, 1)
    gwait(0, 0)
    scale(0, 0)
    scatter(0, 0)
    swait(3, 1)
    swait(0, 0)
    plsc.subcore_barrier()

    pltpu.sync_copy(
        hsh.at[pl.ds(sid * ROWS_PER_SUB, ROWS_PER_SUB)],
        out_hbm.at[cid, pl.ds(sid * ROWS_PER_SUB, ROWS_PER_SUB)])

  return agg


_sc_aggregate = _sc_aggregate_fn()

BLK = 400


def _tc_linear_body(p_ref, w_ref, b_ref, o_ref):
  h = p_ref[0] + p_ref[1]
  acc = lax.dot_general(h, w_ref[...], (((1,), (1,)), ((), ())),
                        preferred_element_type=jnp.float32)
  o_ref[...] = ALPHA * (acc + b_ref[...])


def _tc_linear(partials, W, b2d):
  return pl.pallas_call(
      _tc_linear_body,
      grid=(N // BLK,),
      in_specs=[
          pl.BlockSpec((NC, BLK, D), lambda i: (0, i, 0)),
          pl.BlockSpec((D, D), lambda i: (0, 0)),
          pl.BlockSpec((1, D), lambda i: (0, 0)),
      ],
      out_specs=pl.BlockSpec((BLK, D), lambda i: (i, 0)),
      out_shape=jax.ShapeDtypeStruct((N, D), jnp.float32),
  )(partials, W, b2d)


@jax.jit
def kernel(x, edge_index, edge_w, W, b):
  src = edge_index[0]
  dst = edge_index[1]
  partials = _sc_aggregate(x, src, dst, edge_w)
  return _tc_linear(partials, W, b.reshape(1, D))
